# baseline (speedup 1.0000x reference)
import functools

import numpy as np
import jax
import jax.numpy as jnp
from jax import lax
from jax.experimental import pallas as pl
from jax.experimental.pallas import tpu as pltpu
from jax.experimental.pallas import tpu_sc as plsc

N = 10000
D = 128
E = 320000
NC = 2
NS = 16
NW = NC * NS
NPAD = 10240
RT = NPAD // NS
CHUNK = 112
ET = 10416
EPAD = NW * ET
NCH = ET // CHUNK

_MESH = dict(core_axis_name="c", subcore_axis_name="s")

_NPADDING = EPAD - E
_PAD_SRC = np.arange(_NPADDING, dtype=np.int32) % N
_PAD_DST = (N + np.arange(_NPADDING, dtype=np.int32) % (NPAD - N)).astype(np.int32)
_ZEROS1 = np.zeros((NPAD,), np.float32)
_ZEROS2 = np.zeros((NPAD, D), np.float32)
_ONES_C = np.ones((CHUNK,), np.float32)


def _sc_degree(dst, zeros1, ones_c):
    @functools.partial(
        pl.kernel,
        mesh=plsc.VectorSubcoreMesh(**_MESH),
        out_type=jax.ShapeDtypeStruct((NC, NPAD), jnp.float32),
        scratch_types=[
            pltpu.VMEM((CHUNK,), jnp.int32),
            pltpu.VMEM((CHUNK,), jnp.int32),
            pltpu.VMEM((CHUNK,), jnp.int32),
            pltpu.VMEM((CHUNK,), jnp.float32),
            pltpu.VMEM_SHARED((NPAD,), jnp.float32),
            pltpu.SemaphoreType.DMA,
            pltpu.SemaphoreType.DMA,
            pltpu.SemaphoreType.DMA,
            pltpu.SemaphoreType.DMA,
            pltpu.SemaphoreType.DMA,
            pltpu.SemaphoreType.DMA,
        ],
    )
    def k(dst_hbm, z1, ones_hbm, out, didx0, didx1, didx2, ones_v, acc,
          dsem0, dsem1, dsem2, ssb0, ssb1, ssb2):
        cid = lax.axis_index("c")
        sid = lax.axis_index("s")
        didx = (didx0, didx1, didx2)
        dsem = (dsem0, dsem1, dsem2)
        ssb = (ssb0, ssb1, ssb2)
        base_rows = sid * RT
        pltpu.sync_copy(z1.at[pl.ds(base_rows, RT)], acc.at[pl.ds(base_rows, RT)])
        pltpu.sync_copy(ones_hbm, ones_v)
        plsc.subcore_barrier()

        tile_base = (cid * NS + sid) * ET

        def idx_copy(j, b):
            base = pl.multiple_of(tile_base + j * CHUNK, 8)
            return pltpu.make_async_copy(
                dst_hbm.at[pl.ds(base, CHUNK)], didx[b], dsem[b])

        def sc(b):
            return pltpu.make_async_copy(ones_v, acc.at[didx[b]], ssb[b])

        idx_copy(0, 0).start()
        idx_copy(1, 1).start()

        def step(j, b0, b2):
            idx_copy(j, b0).wait()
            pltpu.async_copy(ones_v, acc.at[didx[b0]], ssb[b0], add=True)
            @pl.when(j >= 1)
            def _():
                sc(b2).wait()
            @pl.when(j + 2 < NCH)
            def _():
                idx_copy(j + 2, b2).start()

        def body(j3, carry):
            step(j3 * 3, 0, 2)
            step(j3 * 3 + 1, 1, 0)
            step(j3 * 3 + 2, 2, 1)
            return carry

        lax.fori_loop(0, NCH // 3, body, 0)
        sc((NCH - 1) % 3).wait()
        plsc.subcore_barrier()
        pltpu.sync_copy(acc.at[pl.ds(base_rows, RT)],
                        out.at[cid, pl.ds(base_rows, RT)])

    return k(dst, zeros1, ones_c)


def _sc_aggregate(g, src, dst, zeros2):
    @functools.partial(
        pl.kernel,
        mesh=plsc.VectorSubcoreMesh(**_MESH),
        out_type=jax.ShapeDtypeStruct((NC, NPAD, D), jnp.float32),
        scratch_types=[
            pltpu.VMEM((CHUNK,), jnp.int32),
            pltpu.VMEM((CHUNK,), jnp.int32),
            pltpu.VMEM((CHUNK,), jnp.int32),
            pltpu.VMEM((CHUNK,), jnp.int32),
            pltpu.VMEM((CHUNK,), jnp.int32),
            pltpu.VMEM((CHUNK,), jnp.int32),
            pltpu.VMEM((CHUNK, D), jnp.float32),
            pltpu.VMEM((CHUNK, D), jnp.float32),
            pltpu.VMEM((CHUNK, D), jnp.float32),
            pltpu.VMEM_SHARED((NPAD, D), jnp.float32),
            pltpu.SemaphoreType.DMA,
            pltpu.SemaphoreType.DMA,
            pltpu.SemaphoreType.DMA,
            pltpu.SemaphoreType.DMA,
            pltpu.SemaphoreType.DMA,
            pltpu.SemaphoreType.DMA,
            pltpu.SemaphoreType.DMA,
            pltpu.SemaphoreType.DMA,
        ],
    )
    def k(g_hbm, src_hbm, dst_hbm, z2, out,
          sidx0, sidx1, sidx2, didx0, didx1, didx2,
          rows0, rows1, rows2, acc,
          isem, dsem0, dsem1, dsem2, gsem0, gsem1, gsem2, ssem):
        cid = lax.axis_index("c")
        sid = lax.axis_index("s")
        sidx = (sidx0, sidx1, sidx2)
        didx = (didx0, didx1, didx2)
        rows = (rows0, rows1, rows2)
        base_rows = sid * RT
        pltpu.sync_copy(z2.at[pl.ds(base_rows, RT), :],
                        acc.at[pl.ds(base_rows, RT), :])
        plsc.subcore_barrier()

        tile_base = (cid * NS + sid) * ET

        def sidx_copy(j, b):
            base = pl.multiple_of(tile_base + j * CHUNK, 8)
            return pltpu.make_async_copy(src_hbm.at[pl.ds(base, CHUNK)],
                                         sidx[b], isem)

        dsem = (dsem0, dsem1, dsem2)
        gsem = (gsem0, gsem1, gsem2)

        def didx_copy(j, b):
            base = pl.multiple_of(tile_base + j * CHUNK, 8)
            return pltpu.make_async_copy(dst_hbm.at[pl.ds(base, CHUNK)],
                                         didx[b], dsem[b])

        def gather(b):
            return pltpu.make_async_copy(g_hbm.at[sidx[b]], rows[b], gsem[b])

        def scatter(b):
            return pltpu.make_async_copy(rows[b], acc.at[didx[b]], ssem)

        c0, c1 = sidx_copy(0, 0), didx_copy(0, 0)
        c0.start(); c1.start(); c0.wait(); c1.wait()
        gather(0).start()
        c0, c1 = sidx_copy(1, 1), didx_copy(1, 1)
        c0.start(); c1.start(); c0.wait(); c1.wait()
        gather(1).start()
        sidx_copy(2, 2).start()

        def step(j, b0, b2):
            @pl.when(j >= 1)
            def _():
                scatter(b2).wait()
            @pl.when(j + 2 < NCH)
            def _():
                sidx_copy(j + 2, b2).wait()
                didx_copy(j + 2, b2).start()
                gather(b2).start()
            gather(b0).wait()
            @pl.when(j + 3 < NCH)
            def _():
                sidx_copy(j + 3, b0).start()
            @pl.when(j >= 2)
            def _():
                didx_copy(j, b0).wait()
            pltpu.async_copy(rows[b0], acc.at[didx[b0]], ssem, add=True)

        def body(j3, carry):
            step(j3 * 3, 0, 2)
            step(j3 * 3 + 1, 1, 0)
            step(j3 * 3 + 2, 2, 1)
            return carry

        lax.fori_loop(0, NCH // 3, body, 0)
        scatter((NCH - 1) % 3).wait()
        plsc.subcore_barrier()
        pltpu.sync_copy(acc.at[pl.ds(base_rows, RT), :],
                        out.at[cid, pl.ds(base_rows, RT), :])

    return k(g, src, dst, zeros2)


def _tc_scale(x, dis_col):
    def body(x_ref, d_ref, o_ref):
        o_ref[...] = x_ref[...] * d_ref[...]

    return pl.pallas_call(
        body, out_shape=jax.ShapeDtypeStruct((N, D), jnp.float32),
    )(x, dis_col)


def _tc_layer(P, g, dis_col, W, b, gamma, beta):
    def body(p_ref, g_ref, d_ref, w_ref, b_ref, ga_ref, be_ref, o_ref):
        s = p_ref[0, :N, :] + p_ref[1, :N, :] + g_ref[...]
        t = s * d_ref[...]
        u = lax.dot_general(t, w_ref[...], (((1,), (0,)), ((), ())),
                            preferred_element_type=jnp.float32,
                            precision=lax.Precision.HIGHEST) + b_ref[...]
        mean = jnp.mean(u, axis=0, keepdims=True)
        var = jnp.mean((u - mean) ** 2, axis=0, keepdims=True)
        v = (u - mean) * lax.rsqrt(var + 1e-5) * ga_ref[...] + be_ref[...]
        v = jnp.maximum(v, 0.0)
        o_ref[...] = v * d_ref[...]

    return pl.pallas_call(
        body, out_shape=jax.ShapeDtypeStruct((N, D), jnp.float32),
    )(P, g, dis_col, W, b, gamma, beta)


def _tc_final(P, g, dis_col, W, b, gamma, beta, lw1, lb1, lw2, lb2):
    def body(p_ref, g_ref, d_ref, w_ref, b_ref, ga_ref, be_ref,
             lw1_ref, lb1_ref, lw2_ref, lb2_ref, out_ref, pool_ref):
        s = p_ref[0, :N, :] + p_ref[1, :N, :] + g_ref[...]
        t = s * d_ref[...]
        u = lax.dot_general(t, w_ref[...], (((1,), (0,)), ((), ())),
                            preferred_element_type=jnp.float32,
                            precision=lax.Precision.HIGHEST) + b_ref[...]
        mean = jnp.mean(u, axis=0, keepdims=True)
        var = jnp.mean((u - mean) ** 2, axis=0, keepdims=True)
        h3 = (u - mean) * lax.rsqrt(var + 1e-5) * ga_ref[...] + be_ref[...]
        h3 = jnp.maximum(h3, 0.0)
        m1 = lax.dot_general(h3, lw1_ref[...], (((1,), (0,)), ((), ())),
                             preferred_element_type=jnp.float32,
                             precision=lax.Precision.HIGHEST) + lb1_ref[...]
        m1 = jnp.maximum(m1, 0.0)
        out_ref[...] = lax.dot_general(m1, lw2_ref[...], (((1,), (0,)), ((), ())),
                                       preferred_element_type=jnp.float32,
                                       precision=lax.Precision.HIGHEST) + lb2_ref[...]
        pool_ref[...] = jnp.mean(h3, axis=0, keepdims=True)

    return pl.pallas_call(
        body,
        out_shape=(jax.ShapeDtypeStruct((N, D), jnp.float32),
                   jax.ShapeDtypeStruct((1, D), jnp.float32)),
    )(P, g, dis_col, W, b, gamma, beta, lw1, lb1, lw2, lb2)


def kernel(x, edge_index, W0, b0, gamma0, beta0, W1, b1, gamma1, beta1,
           W2, b2, gamma2, beta2, lw1, lb1, lw2, lb2):
    zeros1 = _ZEROS1
    zeros2 = _ZEROS2
    ones_c = _ONES_C
    src = jnp.concatenate([edge_index[0], _PAD_SRC])
    dst = jnp.concatenate([edge_index[1], _PAD_DST])

    degp = _sc_degree(dst, zeros1, ones_c)
    deg = degp[0, :N] + degp[1, :N] + 1.0
    dis_col = lax.rsqrt(deg).reshape(N, 1)

    b0r, g0r, be0 = b0.reshape(1, D), gamma0.reshape(1, D), beta0.reshape(1, D)
    b1r, g1r, be1 = b1.reshape(1, D), gamma1.reshape(1, D), beta1.reshape(1, D)
    b2r, g2r, be2 = b2.reshape(1, D), gamma2.reshape(1, D), beta2.reshape(1, D)

    g = _tc_scale(x, dis_col)
    P = _sc_aggregate(g, src, dst, zeros2)
    g = _tc_layer(P, g, dis_col, W0, b0r, g0r, be0)
    P = _sc_aggregate(g, src, dst, zeros2)
    g = _tc_layer(P, g, dis_col, W1, b1r, g1r, be1)
    P = _sc_aggregate(g, src, dst, zeros2)
    out, pooled = _tc_final(P, g, dis_col, W2, b2r, g2r, be2,
                            lw1.reshape(D, D), lb1.reshape(1, D),
                            lw2.reshape(D, D), lb2.reshape(1, D))
    return (out, pooled)


# speedup vs baseline: 1.0182x; 1.0182x over previous
import functools

import numpy as np
import jax
import jax.numpy as jnp
from jax import lax
from jax.experimental import pallas as pl
from jax.experimental.pallas import tpu as pltpu
from jax.experimental.pallas import tpu_sc as plsc

N = 10000
D = 128
E = 320000
NC = 2
NS = 16
NW = NC * NS
NPAD = 10240
RT = NPAD // NS
CHUNK = 112
ET = 10416
EPAD = NW * ET
NCH = ET // CHUNK

_MESH = dict(core_axis_name="c", subcore_axis_name="s")

CHUNK_D = 200
ET_D = E // NW
NCH_D = ET_D // CHUNK_D

_NPADDING = EPAD - E
_PAD_SRC = np.arange(_NPADDING, dtype=np.int32) % N
_PAD_DST = (N + np.arange(_NPADDING, dtype=np.int32) % (NPAD - N)).astype(np.int32)
_ZEROS1 = np.zeros((NPAD,), np.float32)
_ZEROS2 = np.zeros((NPAD, D), np.float32)
_ONES_C = np.ones((CHUNK_D,), np.float32)


def _sc_degree(dst, zeros1, ones_c):
    @functools.partial(
        pl.kernel,
        mesh=plsc.VectorSubcoreMesh(**_MESH),
        out_type=jax.ShapeDtypeStruct((NC, NPAD), jnp.float32),
        scratch_types=[
            pltpu.VMEM((CHUNK_D,), jnp.int32),
            pltpu.VMEM((CHUNK_D,), jnp.int32),
            pltpu.VMEM((CHUNK_D,), jnp.int32),
            pltpu.VMEM((CHUNK_D,), jnp.float32),
            pltpu.VMEM_SHARED((NPAD,), jnp.float32),
            pltpu.SemaphoreType.DMA,
            pltpu.SemaphoreType.DMA,
            pltpu.SemaphoreType.DMA,
            pltpu.SemaphoreType.DMA,
            pltpu.SemaphoreType.DMA,
            pltpu.SemaphoreType.DMA,
        ],
    )
    def k(dst_hbm, z1, ones_hbm, out, didx0, didx1, didx2, ones_v, acc,
          dsem0, dsem1, dsem2, ssb0, ssb1, ssb2):
        cid = lax.axis_index("c")
        sid = lax.axis_index("s")
        didx = (didx0, didx1, didx2)
        dsem = (dsem0, dsem1, dsem2)
        ssb = (ssb0, ssb1, ssb2)
        base_rows = sid * RT
        pltpu.sync_copy(z1.at[pl.ds(base_rows, RT)], acc.at[pl.ds(base_rows, RT)])
        pltpu.sync_copy(ones_hbm, ones_v)
        plsc.subcore_barrier()

        tile_base = (cid * NS + sid) * ET_D

        def idx_copy(j, b):
            base = pl.multiple_of(tile_base + j * CHUNK_D, 8)
            return pltpu.make_async_copy(
                dst_hbm.at[pl.ds(base, CHUNK_D)], didx[b], dsem[b])

        def sc(b):
            return pltpu.make_async_copy(ones_v, acc.at[didx[b]], ssb[b])

        idx_copy(0, 0).start()
        idx_copy(1, 1).start()

        def step(j, b0, b2):
            idx_copy(j, b0).wait()
            pltpu.async_copy(ones_v, acc.at[didx[b0]], ssb[b0], add=True)
            @pl.when(j >= 1)
            def _():
                sc(b2).wait()
            @pl.when(j + 2 < NCH_D)
            def _():
                idx_copy(j + 2, b2).start()

        def body(j3, carry):
            step(j3 * 3, 0, 2)
            step(j3 * 3 + 1, 1, 0)
            step(j3 * 3 + 2, 2, 1)
            return carry

        lax.fori_loop(0, (NCH_D - 2) // 3, body, 0)
        step(NCH_D - 2, 0, 2)
        step(NCH_D - 1, 1, 0)
        sc((NCH_D - 1) % 3).wait()
        plsc.subcore_barrier()
        pltpu.sync_copy(acc.at[pl.ds(base_rows, RT)],
                        out.at[cid, pl.ds(base_rows, RT)])

    return k(dst, zeros1, ones_c)


def _sc_aggregate(g, src, dst, zeros2):
    @functools.partial(
        pl.kernel,
        mesh=plsc.VectorSubcoreMesh(**_MESH),
        out_type=jax.ShapeDtypeStruct((NC, NPAD, D), jnp.float32),
        scratch_types=[
            pltpu.VMEM((CHUNK,), jnp.int32),
            pltpu.VMEM((CHUNK,), jnp.int32),
            pltpu.VMEM((CHUNK,), jnp.int32),
            pltpu.VMEM((CHUNK,), jnp.int32),
            pltpu.VMEM((CHUNK,), jnp.int32),
            pltpu.VMEM((CHUNK,), jnp.int32),
            pltpu.VMEM((CHUNK, D), jnp.float32),
            pltpu.VMEM((CHUNK, D), jnp.float32),
            pltpu.VMEM((CHUNK, D), jnp.float32),
            pltpu.VMEM_SHARED((NPAD, D), jnp.float32),
            pltpu.SemaphoreType.DMA,
            pltpu.SemaphoreType.DMA,
            pltpu.SemaphoreType.DMA,
            pltpu.SemaphoreType.DMA,
            pltpu.SemaphoreType.DMA,
            pltpu.SemaphoreType.DMA,
            pltpu.SemaphoreType.DMA,
            pltpu.SemaphoreType.DMA,
        ],
    )
    def k(g_hbm, src_hbm, dst_hbm, z2, out,
          sidx0, sidx1, sidx2, didx0, didx1, didx2,
          rows0, rows1, rows2, acc,
          isem, dsem0, dsem1, dsem2, gsem0, gsem1, gsem2, ssem):
        cid = lax.axis_index("c")
        sid = lax.axis_index("s")
        sidx = (sidx0, sidx1, sidx2)
        didx = (didx0, didx1, didx2)
        rows = (rows0, rows1, rows2)
        base_rows = sid * RT
        pltpu.sync_copy(z2.at[pl.ds(base_rows, RT), :],
                        acc.at[pl.ds(base_rows, RT), :])
        plsc.subcore_barrier()

        tile_base = (cid * NS + sid) * ET

        def sidx_copy(j, b):
            base = pl.multiple_of(tile_base + j * CHUNK, 8)
            return pltpu.make_async_copy(src_hbm.at[pl.ds(base, CHUNK)],
                                         sidx[b], isem)

        dsem = (dsem0, dsem1, dsem2)
        gsem = (gsem0, gsem1, gsem2)

        def didx_copy(j, b):
            base = pl.multiple_of(tile_base + j * CHUNK, 8)
            return pltpu.make_async_copy(dst_hbm.at[pl.ds(base, CHUNK)],
                                         didx[b], dsem[b])

        def gather(b):
            return pltpu.make_async_copy(g_hbm.at[sidx[b]], rows[b], gsem[b])

        def scatter(b):
            return pltpu.make_async_copy(rows[b], acc.at[didx[b]], ssem)

        c0, c1 = sidx_copy(0, 0), didx_copy(0, 0)
        c0.start(); c1.start(); c0.wait(); c1.wait()
        gather(0).start()
        c0, c1 = sidx_copy(1, 1), didx_copy(1, 1)
        c0.start(); c1.start(); c0.wait(); c1.wait()
        gather(1).start()
        sidx_copy(2, 2).start()

        def step(j, b0, b2):
            @pl.when(j >= 1)
            def _():
                scatter(b2).wait()
            @pl.when(j + 2 < NCH)
            def _():
                sidx_copy(j + 2, b2).wait()
                didx_copy(j + 2, b2).start()
                gather(b2).start()
            gather(b0).wait()
            @pl.when(j + 3 < NCH)
            def _():
                sidx_copy(j + 3, b0).start()
            @pl.when(j >= 2)
            def _():
                didx_copy(j, b0).wait()
            pltpu.async_copy(rows[b0], acc.at[didx[b0]], ssem, add=True)

        def body(j3, carry):
            step(j3 * 3, 0, 2)
            step(j3 * 3 + 1, 1, 0)
            step(j3 * 3 + 2, 2, 1)
            return carry

        lax.fori_loop(0, NCH // 3, body, 0)
        scatter((NCH - 1) % 3).wait()
        plsc.subcore_barrier()
        pltpu.sync_copy(acc.at[pl.ds(base_rows, RT), :],
                        out.at[cid, pl.ds(base_rows, RT), :])

    return k(g, src, dst, zeros2)


def _tc_scale(x, dis_col):
    def body(x_ref, d_ref, o_ref):
        o_ref[...] = x_ref[...] * d_ref[...]

    return pl.pallas_call(
        body, out_shape=jax.ShapeDtypeStruct((N, D), jnp.float32),
    )(x, dis_col)


def _tc_layer(P, g, dis_col, W, b, gamma, beta):
    def body(p_ref, g_ref, d_ref, w_ref, b_ref, ga_ref, be_ref, o_ref):
        s = p_ref[0, :N, :] + p_ref[1, :N, :] + g_ref[...]
        t = s * d_ref[...]
        u = lax.dot_general(t, w_ref[...], (((1,), (0,)), ((), ())),
                            preferred_element_type=jnp.float32,
                            precision=lax.Precision.HIGHEST) + b_ref[...]
        mean = jnp.mean(u, axis=0, keepdims=True)
        var = jnp.mean((u - mean) ** 2, axis=0, keepdims=True)
        v = (u - mean) * lax.rsqrt(var + 1e-5) * ga_ref[...] + be_ref[...]
        v = jnp.maximum(v, 0.0)
        o_ref[...] = v * d_ref[...]

    return pl.pallas_call(
        body, out_shape=jax.ShapeDtypeStruct((N, D), jnp.float32),
    )(P, g, dis_col, W, b, gamma, beta)


def _tc_final(P, g, dis_col, W, b, gamma, beta, lw1, lb1, lw2, lb2):
    def body(p_ref, g_ref, d_ref, w_ref, b_ref, ga_ref, be_ref,
             lw1_ref, lb1_ref, lw2_ref, lb2_ref, out_ref, pool_ref):
        s = p_ref[0, :N, :] + p_ref[1, :N, :] + g_ref[...]
        t = s * d_ref[...]
        u = lax.dot_general(t, w_ref[...], (((1,), (0,)), ((), ())),
                            preferred_element_type=jnp.float32,
                            precision=lax.Precision.HIGHEST) + b_ref[...]
        mean = jnp.mean(u, axis=0, keepdims=True)
        var = jnp.mean((u - mean) ** 2, axis=0, keepdims=True)
        h3 = (u - mean) * lax.rsqrt(var + 1e-5) * ga_ref[...] + be_ref[...]
        h3 = jnp.maximum(h3, 0.0)
        m1 = lax.dot_general(h3, lw1_ref[...], (((1,), (0,)), ((), ())),
                             preferred_element_type=jnp.float32,
                             precision=lax.Precision.HIGHEST) + lb1_ref[...]
        m1 = jnp.maximum(m1, 0.0)
        out_ref[...] = lax.dot_general(m1, lw2_ref[...], (((1,), (0,)), ((), ())),
                                       preferred_element_type=jnp.float32,
                                       precision=lax.Precision.HIGHEST) + lb2_ref[...]
        pool_ref[...] = jnp.mean(h3, axis=0, keepdims=True)

    return pl.pallas_call(
        body,
        out_shape=(jax.ShapeDtypeStruct((N, D), jnp.float32),
                   jax.ShapeDtypeStruct((1, D), jnp.float32)),
    )(P, g, dis_col, W, b, gamma, beta, lw1, lb1, lw2, lb2)


def kernel(x, edge_index, W0, b0, gamma0, beta0, W1, b1, gamma1, beta1,
           W2, b2, gamma2, beta2, lw1, lb1, lw2, lb2):
    zeros1 = _ZEROS1
    zeros2 = _ZEROS2
    ones_c = _ONES_C
    src = jnp.concatenate([edge_index[0], _PAD_SRC])
    dst = jnp.concatenate([edge_index[1], _PAD_DST])
    dst_raw = lax.optimization_barrier(edge_index[1])

    degp = _sc_degree(dst_raw, zeros1, ones_c)
    deg = degp[0, :N] + degp[1, :N] + 1.0
    dis_col = lax.rsqrt(deg).reshape(N, 1)

    b0r, g0r, be0 = b0.reshape(1, D), gamma0.reshape(1, D), beta0.reshape(1, D)
    b1r, g1r, be1 = b1.reshape(1, D), gamma1.reshape(1, D), beta1.reshape(1, D)
    b2r, g2r, be2 = b2.reshape(1, D), gamma2.reshape(1, D), beta2.reshape(1, D)

    g = _tc_scale(x, dis_col)
    P = _sc_aggregate(g, src, dst, zeros2)
    g = _tc_layer(P, g, dis_col, W0, b0r, g0r, be0)
    P = _sc_aggregate(g, src, dst, zeros2)
    g = _tc_layer(P, g, dis_col, W1, b1r, g1r, be1)
    P = _sc_aggregate(g, src, dst, zeros2)
    out, pooled = _tc_final(P, g, dis_col, W2, b2r, g2r, be2,
                            lw1.reshape(D, D), lb1.reshape(1, D),
                            lw2.reshape(D, D), lb2.reshape(1, D))
    return (out, pooled)
